# Initial kernel scaffold; baseline (speedup 1.0000x reference)
import jax
import jax.numpy as jnp
from jax.experimental import pallas as pl


def kernel(sample, table, Wk, q, Wg, bg, Wt, bt, use_text_moe):
    raise NotImplementedError("write your pallas kernel here")


# speedup vs baseline: 4.6297x; 4.6297x over previous
import functools

import jax
import jax.numpy as jnp
from jax import lax
from jax.experimental import pallas as pl
from jax.experimental.pallas import tpu as pltpu
from jax.experimental.pallas import tpu_sc as plsc

B = 4096
T = 50
V = 100000
D = 128
E = 16
K = 2
F_OUT = 128
PAD = 0

N_IDX = B * T
NW = 32
B_PER_W = N_IDX // NW
CHUNK = 640
N_CHUNKS = B_PER_W // CHUNK


def _sc_gather(idx_hbm, table_hbm, out_hbm, idx_v, rows_v, sem):
    wid = lax.axis_index("s") * 2 + lax.axis_index("c")
    base = wid * B_PER_W
    pltpu.sync_copy(idx_hbm.at[pl.ds(base, B_PER_W)], idx_v)

    def body(c, _):
        off = c * CHUNK
        pltpu.async_copy(table_hbm.at[idx_v.at[pl.ds(off, CHUNK)]], rows_v,
                         sem).wait()
        pltpu.sync_copy(rows_v, out_hbm.at[pl.ds(base + off, CHUNK)])
        return ()

    lax.fori_loop(0, N_CHUNKS, body, (), unroll=False)


def _gather_tokens(sample, table):
    idx = sample.reshape(N_IDX).astype(jnp.int32)
    mesh = plsc.VectorSubcoreMesh(core_axis_name="c", subcore_axis_name="s")
    k = functools.partial(
        pl.kernel,
        mesh=mesh,
        out_type=jax.ShapeDtypeStruct((N_IDX, D), jnp.float32),
        scratch_types=[
            pltpu.VMEM((B_PER_W,), jnp.int32),
            pltpu.VMEM((CHUNK, D), jnp.float32),
            pltpu.SemaphoreType.DMA,
        ],
    )(_sc_gather)
    return k(idx, table)


R = 256


def _tc_moe(sample_ref, tok_ref, wk_ref, q_ref, wg_ref, bg_ref, wt_ref,
            bt_ref, out_ref):
    proj = jnp.concatenate(
        [jax.lax.dot_general(q_ref[pl.ds(e, 1), :], wk_ref[e],
                             (((1,), (0,)), ((), ())),
                             preferred_element_type=jnp.float32)
         for e in range(E)], axis=0)

    tok = tok_ref[...]
    mask3 = sample_ref[...] == PAD
    tok = jnp.where(mask3, 0.0, tok)

    flat = tok.reshape(R * T, D)
    s = jax.lax.dot_general(flat, proj, (((1,), (1,)), ((), ())),
                            preferred_element_type=jnp.float32)
    s = s.reshape(R, T, E)
    s = jnp.where(mask3, -1e30, s)
    s = s - jnp.max(s, axis=1, keepdims=True)
    es = jnp.exp(s)
    attn = es / jnp.sum(es, axis=1, keepdims=True)

    gate_in = jnp.mean(tok, axis=1)
    g = jax.lax.dot_general(gate_in, wg_ref[...], (((1,), (1,)), ((), ())),
                            preferred_element_type=jnp.float32)
    g = g + bg_ref[...]

    eids = lax.broadcasted_iota(jnp.int32, (R, E), 1)
    m1 = jnp.max(g, axis=1, keepdims=True)
    i1 = jnp.min(jnp.where(g == m1, eids, E), axis=1, keepdims=True)
    g2 = jnp.where(eids == i1, -jnp.inf, g)
    m2 = jnp.max(g2, axis=1, keepdims=True)
    i2 = jnp.min(jnp.where(g2 == m2, eids, E), axis=1, keepdims=True)
    e2 = jnp.exp(m2 - m1)
    w1 = 1.0 / (1.0 + e2)
    w2 = e2 * w1
    full = jnp.where(eids == i1, w1, 0.0) + jnp.where(eids == i2, w2, 0.0)

    wts3 = jnp.sum(attn * full[:, None, :], axis=2, keepdims=True)
    pooled = jnp.sum(tok * wts3, axis=1)

    z = jax.lax.dot_general(pooled, wt_ref[...], (((1,), (1,)), ((), ())),
                            preferred_element_type=jnp.float32)
    z = z + bt_ref[...]
    out_ref[...] = z * jax.nn.sigmoid(z)


def _moe_dense(sample3, tok, Wk, q, Wg, bg, Wt, bt):
    grid = (B // R,)
    return pl.pallas_call(
        _tc_moe,
        grid=grid,
        in_specs=[
            pl.BlockSpec((R, T, 1), lambda i: (i, 0, 0)),
            pl.BlockSpec((R, T, D), lambda i: (i, 0, 0)),
            pl.BlockSpec((E, D, D), lambda i: (0, 0, 0)),
            pl.BlockSpec((E, D), lambda i: (0, 0)),
            pl.BlockSpec((E, D), lambda i: (0, 0)),
            pl.BlockSpec((1, E), lambda i: (0, 0)),
            pl.BlockSpec((F_OUT, D), lambda i: (0, 0)),
            pl.BlockSpec((1, F_OUT), lambda i: (0, 0)),
        ],
        out_specs=pl.BlockSpec((R, F_OUT), lambda i: (i, 0)),
        out_shape=jax.ShapeDtypeStruct((B, F_OUT), jnp.float32),
    )(sample3, tok, Wk, q, Wg, bg, Wt, bt)


def kernel(sample, table, Wk, q, Wg, bg, Wt, bt, use_text_moe):
    tok_flat = _gather_tokens(sample, table)
    tok = tok_flat.reshape(B, T, D)
    out = _moe_dense(sample.astype(jnp.int32).reshape(B, T, 1), tok,
                     Wk, q, Wg, bg.reshape(1, E), Wt, bt.reshape(1, F_OUT))
    return out * jnp.asarray(use_text_moe, out.dtype)
